# baseline (speedup 1.0000x reference)
import functools

import jax
import jax.numpy as jnp
from jax import lax
from jax.experimental import pallas as pl
from jax.experimental.pallas import tpu as pltpu
from jax.experimental.pallas import tpu_sc as plsc

_NC = 2
_NS = 16
_NW = _NC * _NS
_CW = 128


@functools.partial(jax.jit, static_argnames=("ch", "d"))
def _sc_gather(idx, table, ch, d):
    mesh = plsc.VectorSubcoreMesh(core_axis_name="c", subcore_axis_name="s")

    @functools.partial(
        pl.kernel,
        mesh=mesh,
        out_type=jax.ShapeDtypeStruct((_NW, ch, _CW, d), jnp.float32),
        scratch_types=[
            pltpu.VMEM((ch, _CW), jnp.int32),
            pltpu.VMEM((ch, _CW, d), jnp.float32),
            pltpu.SemaphoreType.DMA,
        ],
        compiler_params=pltpu.CompilerParams(use_tc_tiling_on_sc=False),
    )
    def k(idx_hbm, table_hbm, out_hbm, idx_v, rows_v, sem):
        wid = lax.axis_index("s") * _NC + lax.axis_index("c")
        pltpu.sync_copy(idx_hbm.at[wid], idx_v)
        copies = [
            pltpu.async_copy(table_hbm.at[idx_v.at[j]], rows_v.at[j], sem)
            for j in range(ch)
        ]
        for c in copies:
            c.wait()
        pltpu.sync_copy(rows_v, out_hbm.at[wid])

    return k(idx, table)


def kernel(batch, table):
    (b,) = batch.shape
    _, d = table.shape
    ch = b // (_NW * _CW)
    idx = batch.astype(jnp.int32).reshape(_NW, ch, _CW)
    out = _sc_gather(idx, table, ch, d)
    return out.reshape(b, 1, d)
